# Initial kernel scaffold; baseline (speedup 1.0000x reference)
import jax
import jax.numpy as jnp
from jax.experimental import pallas as pl


def kernel(x, edge_index, type_logit, col_logit, vib_mu_W, vib_mu_b, vib_lv_W, vib_lv_b, vib_eps, W_self1, W_neigh1, W_self2, W_neigh2, W_head, b_head):
    raise NotImplementedError("write your pallas kernel here")


# speedup vs baseline: 8.3300x; 8.3300x over previous
import jax
import jax.numpy as jnp
from jax import lax
from jax.experimental import pallas as pl
from jax.experimental.pallas import tpu as pltpu
from jax.experimental.pallas import tpu_sc as plsc

N = 10000
E = 320000
D = 128
OUT = 128
TAU = 2.0
GAMMA = -0.1
ZETA = 1.1

NC = 2
NS = 16
NW = NC * NS
EW = E // NW
CH = 50
NCH = EW // CH
G = 8
NG = NCH // G
RT = 624
ZR = 8

ROWS = 1000
GRID = N // ROWS

_f32 = jnp.float32


def _gate(logit):
    s = jax.nn.sigmoid(logit / TAU)
    return jnp.clip(s * (ZETA - GAMMA) + GAMMA, 0.0, 1.0)


def _dot(a, b):
    return lax.dot_general(a, b, (((1,), (0,)), ((), ())),
                           precision=lax.Precision.HIGHEST,
                           preferred_element_type=_f32)


def _vib_body(x_ref, eps_ref, tl_ref, cl_ref, muW_ref, mub_ref, lvW_ref,
              lvb_ref, z_ref, kl_ref):
    i = pl.program_id(0)
    g = _gate(cl_ref[...]) * _gate(tl_ref[...])
    h = x_ref[...] * g
    mu = _dot(h, muW_ref[...]) + mub_ref[...]
    lv = jnp.clip(_dot(h, lvW_ref[...]) + lvb_ref[...], -10.0, 10.0)
    elv = jnp.exp(lv)
    z_ref[...] = mu + eps_ref[...] * jnp.exp(0.5 * lv)
    part = jnp.sum(elv + mu * mu - 1.0 - lv) * (0.5 / N)

    @pl.when(i == 0)
    def _():
        kl_ref[...] = jnp.zeros((1, 1), _f32)

    kl_ref[...] += jnp.reshape(part, (1, 1))


def _vib_call(x, eps, tl, cl, muW, mub, lvW, lvb):
    full = lambda s: pl.BlockSpec(s, lambda i: (0,) * len(s))
    return pl.pallas_call(
        _vib_body,
        grid=(GRID,),
        in_specs=[
            pl.BlockSpec((ROWS, D), lambda i: (i, 0)),
            pl.BlockSpec((ROWS, D), lambda i: (i, 0)),
            full((1, 1)), full((1, D)), full((D, D)), full((1, D)),
            full((D, D)), full((1, D)),
        ],
        out_specs=[
            pl.BlockSpec((ROWS, D), lambda i: (i, 0)),
            pl.BlockSpec((1, 1), lambda i: (0, 0)),
        ],
        out_shape=[
            jax.ShapeDtypeStruct((N, D), _f32),
            jax.ShapeDtypeStruct((1, 1), _f32),
        ],
    )(x, eps, tl, cl, muW, mub, lvW, lvb)


def _sc_body(table, src2d, dst2d, msg_out,
             sg0, dg0, sg1, dg1, rows_a, rows_b, zb, acc,
             semi0, semi1, sem_a, sem_b):
    cid = lax.axis_index("c")
    sid = lax.axis_index("s")
    wid = cid * NS + sid
    row0 = sid * RT
    base = wid * NCH

    @pl.loop(0, ZR)
    def _(r):
        @pl.loop(0, D // 16)
        def _(c):
            zb[r, pl.ds(c * 16, 16)] = jnp.zeros((16,), _f32)

    @pl.loop(0, RT // ZR)
    def _(r):
        pltpu.sync_copy(zb, acc.at[pl.ds(row0 + r * ZR, ZR)])

    @pl.when(sid == NS - 1)
    def _():
        @pl.loop(RT // ZR, RT // ZR + 2)
        def _(r):
            pltpu.sync_copy(zb, acc.at[pl.ds(row0 + r * ZR, ZR)])

    plsc.subcore_barrier()

    bufs = (rows_a, rows_b)
    sems = (sem_a, sem_b)

    def idx_load(grp, sg, dg, sem):
        off = pl.multiple_of(base + grp * G, 8)
        pltpu.async_copy(src2d.at[pl.ds(off, G)], sg, sem)
        pltpu.async_copy(dst2d.at[pl.ds(off, G)], dg, sem)

    def idx_wait(grp, sg, dg, sem):
        off = pl.multiple_of(base + grp * G, 8)
        pltpu.make_async_copy(src2d.at[pl.ds(off, G)], sg, sem).wait()
        pltpu.make_async_copy(dst2d.at[pl.ds(off, G)], dg, sem).wait()

    def gather(sg, jj, par):
        pltpu.async_copy(table.at[sg.at[jj]], bufs[par], sems[par])

    def process(sg, dg, jj, par):
        pltpu.make_async_copy(table.at[sg.at[jj]], bufs[par],
                              sems[par]).wait()
        pltpu.sync_copy(bufs[par], acc.at[dg.at[jj]], add=True)

    def group_body(g, sgc, dgc, semic, sgn, dgn, semin):
        for jj in range(G):
            par = jj & 1
            if jj < G - 1:
                gather(sgc, jj + 1, 1 - par)
            else:
                @pl.when(g < NG - 1)
                def _():
                    idx_wait(g + 1, sgn, dgn, semin)
                    gather(sgn, 0, 0)
            process(sgc, dgc, jj, par)

        @pl.when(g + 2 < NG)
        def _():
            idx_load(g + 2, sgc, dgc, semic)

    off0 = pl.multiple_of(base, 8)
    pltpu.sync_copy(src2d.at[pl.ds(off0, G)], sg0)
    pltpu.sync_copy(dst2d.at[pl.ds(off0, G)], dg0)
    idx_load(1, sg1, dg1, semi1)
    gather(sg0, 0, 0)

    @pl.loop(0, NG)
    def _(g):
        @pl.when(g % 2 == 0)
        def _():
            group_body(g, sg0, dg0, semi0, sg1, dg1, semi1)

        @pl.when(g % 2 == 1)
        def _():
            group_body(g, sg1, dg1, semi1, sg0, dg0, semi0)

    plsc.subcore_barrier()

    pltpu.sync_copy(acc.at[pl.ds(row0, RT)], msg_out.at[cid, pl.ds(row0, RT)])

    @pl.when(sid == NS - 1)
    def _():
        pltpu.sync_copy(acc.at[pl.ds(N - 16, 16)],
                        msg_out.at[cid, pl.ds(N - 16, 16)])


def _mesh():
    return plsc.VectorSubcoreMesh(core_axis_name="c", subcore_axis_name="s",
                                  num_cores=NC, num_subcores=NS)


def _sc_call(table, src2d, dst2d):
    f = pl.kernel(
        _sc_body,
        out_type=[jax.ShapeDtypeStruct((NC, N, D), _f32)],
        mesh=_mesh(),
        scratch_types=[
            pltpu.VMEM((G, CH), jnp.int32),
            pltpu.VMEM((G, CH), jnp.int32),
            pltpu.VMEM((G, CH), jnp.int32),
            pltpu.VMEM((G, CH), jnp.int32),
            pltpu.VMEM((CH, D), _f32),
            pltpu.VMEM((CH, D), _f32),
            pltpu.VMEM((ZR, D), _f32),
            pltpu.VMEM_SHARED((N, D), _f32),
            pltpu.SemaphoreType.DMA,
            pltpu.SemaphoreType.DMA,
            pltpu.SemaphoreType.DMA,
            pltpu.SemaphoreType.DMA,
        ],
    )
    return f(table, src2d, dst2d)[0]


def _deg_body(dst1d, deg_out, dgf, degv):
    cid = lax.axis_index("c")
    sid = lax.axis_index("s")
    wid = cid * NS + sid

    @pl.loop(0, N // 16)
    def _(i):
        degv[pl.ds(i * 16, 16)] = jnp.zeros((16,), _f32)

    @pl.loop(0, NG)
    def _(g):
        off = pl.multiple_of(wid * EW + g * G * CH, 8)
        pltpu.sync_copy(dst1d.at[pl.ds(off, G * CH)], dgf)

        @pl.loop(0, G * CH // 16)
        def _(i):
            v = dgf[pl.ds(i * 16, 16)]
            plsc.addupdate_scatter(degv, [v], jnp.ones((16,), _f32))

    pltpu.sync_copy(degv, deg_out.at[wid])


def _deg_call(dst1d):
    import dataclasses as _dc
    cp = pltpu.CompilerParams()
    if "needs_layout_passes" in pltpu.CompilerParams.__dataclass_fields__:
        cp = _dc.replace(cp, needs_layout_passes=False)
    f = pl.kernel(
        _deg_body,
        out_type=[jax.ShapeDtypeStruct((NW, N), _f32)],
        mesh=_mesh(),
        compiler_params=cp,
        scratch_types=[
            pltpu.VMEM((G * CH,), jnp.int32),
            pltpu.VMEM((N,), _f32),
        ],
    )
    return f(dst1d)[0]


def _inv_deg(dg_block):
    deg = jnp.clip(jnp.sum(dg_block, axis=1, keepdims=True), 1.0, None)
    return 1.0 / deg


def _l1_body(z_ref, mp_ref, dg_ref, ws_ref, wn_ref, h1_ref):
    mp = mp_ref[...]
    msg = (mp[0] + mp[1]) * _inv_deg(dg_ref[...])
    h1_ref[...] = jnp.maximum(
        _dot(z_ref[...], ws_ref[...]) + _dot(msg, wn_ref[...]), 0.0)


def _l1_call(z, mp, dg, ws, wn):
    full = lambda s: pl.BlockSpec(s, lambda i: (0,) * len(s))
    return pl.pallas_call(
        _l1_body,
        grid=(GRID,),
        in_specs=[
            pl.BlockSpec((ROWS, D), lambda i: (i, 0)),
            pl.BlockSpec((NC, ROWS, D), lambda i: (0, i, 0)),
            pl.BlockSpec((ROWS, NW), lambda i: (i, 0)),
            full((D, D)), full((D, D)),
        ],
        out_specs=pl.BlockSpec((ROWS, D), lambda i: (i, 0)),
        out_shape=jax.ShapeDtypeStruct((N, D), _f32),
    )(z, mp, dg, ws, wn)


def _l2_body(h1_ref, mp_ref, dg_ref, ws_ref, wn_ref, wh_ref, bh_ref,
             out_ref):
    mp = mp_ref[...]
    msg = (mp[0] + mp[1]) * _inv_deg(dg_ref[...])
    h2 = _dot(h1_ref[...], ws_ref[...]) + _dot(msg, wn_ref[...])
    out_ref[...] = _dot(h2, wh_ref[...]) + bh_ref[...]


def _l2_call(h1, mp, dg, ws, wn, wh, bh):
    full = lambda s: pl.BlockSpec(s, lambda i: (0,) * len(s))
    return pl.pallas_call(
        _l2_body,
        grid=(GRID,),
        in_specs=[
            pl.BlockSpec((ROWS, D), lambda i: (i, 0)),
            pl.BlockSpec((NC, ROWS, D), lambda i: (0, i, 0)),
            pl.BlockSpec((ROWS, NW), lambda i: (i, 0)),
            full((D, D)), full((D, D)), full((D, OUT)), full((1, OUT)),
        ],
        out_specs=pl.BlockSpec((ROWS, OUT), lambda i: (i, 0)),
        out_shape=jax.ShapeDtypeStruct((N, OUT), _f32),
    )(h1, mp, dg, ws, wn, wh, bh)


def kernel(x, edge_index, type_logit, col_logit, vib_mu_W, vib_mu_b,
           vib_lv_W, vib_lv_b, vib_eps, W_self1, W_neigh1, W_self2,
           W_neigh2, W_head, b_head):
    src2d = edge_index[0].reshape(NW * NCH, CH)
    dst1d = edge_index[1]
    dst2d = dst1d.reshape(NW * NCH, CH)
    z, kl11 = _vib_call(x, vib_eps, type_logit.reshape(1, 1),
                        col_logit.reshape(1, D), vib_mu_W,
                        vib_mu_b.reshape(1, D), vib_lv_W,
                        vib_lv_b.reshape(1, D))
    degp = _deg_call(dst1d)
    msg1p = _sc_call(z, src2d, dst2d)
    degT = degp.T
    h1 = _l1_call(z, msg1p, degT, W_self1, W_neigh1)
    msg2p = _sc_call(h1, src2d, dst2d)
    out = _l2_call(h1, msg2p, degT, W_self2, W_neigh2, W_head,
                   b_head.reshape(1, OUT))
    return (out, kl11[0, 0])


# speedup vs baseline: 9.0274x; 1.0837x over previous
import jax
import jax.numpy as jnp
from jax import lax
from jax.experimental import pallas as pl
from jax.experimental.pallas import tpu as pltpu
from jax.experimental.pallas import tpu_sc as plsc

N = 10000
E = 320000
D = 128
OUT = 128
TAU = 2.0
GAMMA = -0.1
ZETA = 1.1

NC = 2
NS = 16
NW = NC * NS
EW = E // NW
CH = 25
NCH = EW // CH
G = 16
NG = NCH // G
NB = 4
RT = 624
ZR = 8

ROWS = 1000
GRID = N // ROWS

_f32 = jnp.float32


def _gate(logit):
    s = jax.nn.sigmoid(logit / TAU)
    return jnp.clip(s * (ZETA - GAMMA) + GAMMA, 0.0, 1.0)


def _dot(a, b):
    return lax.dot_general(a, b, (((1,), (0,)), ((), ())),
                           precision=lax.Precision.HIGHEST,
                           preferred_element_type=_f32)


def _vib_body(x_ref, eps_ref, tl_ref, cl_ref, muW_ref, mub_ref, lvW_ref,
              lvb_ref, z_ref, kl_ref):
    i = pl.program_id(0)
    g = _gate(cl_ref[...]) * _gate(tl_ref[...])
    h = x_ref[...] * g
    mu = _dot(h, muW_ref[...]) + mub_ref[...]
    lv = jnp.clip(_dot(h, lvW_ref[...]) + lvb_ref[...], -10.0, 10.0)
    elv = jnp.exp(lv)
    z_ref[...] = mu + eps_ref[...] * jnp.exp(0.5 * lv)
    part = jnp.sum(elv + mu * mu - 1.0 - lv) * (0.5 / N)

    @pl.when(i == 0)
    def _():
        kl_ref[...] = jnp.zeros((1, 1), _f32)

    kl_ref[...] += jnp.reshape(part, (1, 1))


def _vib_call(x, eps, tl, cl, muW, mub, lvW, lvb):
    full = lambda s: pl.BlockSpec(s, lambda i: (0,) * len(s))
    return pl.pallas_call(
        _vib_body,
        grid=(GRID,),
        in_specs=[
            pl.BlockSpec((ROWS, D), lambda i: (i, 0)),
            pl.BlockSpec((ROWS, D), lambda i: (i, 0)),
            full((1, 1)), full((1, D)), full((D, D)), full((1, D)),
            full((D, D)), full((1, D)),
        ],
        out_specs=[
            pl.BlockSpec((ROWS, D), lambda i: (i, 0)),
            pl.BlockSpec((1, 1), lambda i: (0, 0)),
        ],
        out_shape=[
            jax.ShapeDtypeStruct((N, D), _f32),
            jax.ShapeDtypeStruct((1, 1), _f32),
        ],
    )(x, eps, tl, cl, muW, mub, lvW, lvb)


def _sc_body(table, src2d, dst2d, msg_out,
             sg0, dg0, sg1, dg1, rows0, rows1, rows2, rows3, zb, acc,
             semi0, semi1, sem0, sem1, sem2, sem3):
    cid = lax.axis_index("c")
    sid = lax.axis_index("s")
    wid = cid * NS + sid
    row0 = sid * RT
    base = wid * NCH

    @pl.loop(0, ZR)
    def _(r):
        @pl.loop(0, D // 16)
        def _(c):
            zb[r, pl.ds(c * 16, 16)] = jnp.zeros((16,), _f32)

    @pl.loop(0, RT // ZR)
    def _(r):
        pltpu.sync_copy(zb, acc.at[pl.ds(row0 + r * ZR, ZR)])

    @pl.when(sid == NS - 1)
    def _():
        @pl.loop(RT // ZR, RT // ZR + 2)
        def _(r):
            pltpu.sync_copy(zb, acc.at[pl.ds(row0 + r * ZR, ZR)])

    plsc.subcore_barrier()

    bufs = (rows0, rows1, rows2, rows3)
    sems = (sem0, sem1, sem2, sem3)

    def idx_load(grp, sg, dg, sem):
        off = pl.multiple_of(base + grp * G, 8)
        pltpu.async_copy(src2d.at[pl.ds(off, G)], sg, sem)
        pltpu.async_copy(dst2d.at[pl.ds(off, G)], dg, sem)

    def idx_wait(grp, sg, dg, sem):
        off = pl.multiple_of(base + grp * G, 8)
        pltpu.make_async_copy(src2d.at[pl.ds(off, G)], sg, sem).wait()
        pltpu.make_async_copy(dst2d.at[pl.ds(off, G)], dg, sem).wait()

    def gather(sg, jj, par):
        pltpu.async_copy(table.at[sg.at[jj]], bufs[par], sems[par])

    def process(sg, dg, jj, par):
        pltpu.make_async_copy(table.at[sg.at[jj]], bufs[par],
                              sems[par]).wait()
        pltpu.sync_copy(bufs[par], acc.at[dg.at[jj]], add=True)

    def group_body(g, sgc, dgc, semic, sgn, dgn, semin):
        for jj in range(G):
            par = jj % NB
            if jj == G - 3:
                @pl.when(g < NG - 1)
                def _():
                    idx_wait(g + 1, sgn, dgn, semin)
            if jj < G - 3:
                gather(sgc, jj + 3, (jj + 3) % NB)
            else:
                @pl.when(g < NG - 1)
                def _():
                    gather(sgn, jj + 3 - G, (jj + 3) % NB)
            process(sgc, dgc, jj, par)

        @pl.when(g + 2 < NG)
        def _():
            idx_load(g + 2, sgc, dgc, semic)

    off0 = pl.multiple_of(base, 8)
    pltpu.sync_copy(src2d.at[pl.ds(off0, G)], sg0)
    pltpu.sync_copy(dst2d.at[pl.ds(off0, G)], dg0)
    idx_load(1, sg1, dg1, semi1)
    gather(sg0, 0, 0)
    gather(sg0, 1, 1)
    gather(sg0, 2, 2)

    @pl.loop(0, NG)
    def _(g):
        @pl.when(g % 2 == 0)
        def _():
            group_body(g, sg0, dg0, semi0, sg1, dg1, semi1)

        @pl.when(g % 2 == 1)
        def _():
            group_body(g, sg1, dg1, semi1, sg0, dg0, semi0)

    plsc.subcore_barrier()

    pltpu.sync_copy(acc.at[pl.ds(row0, RT)], msg_out.at[cid, pl.ds(row0, RT)])

    @pl.when(sid == NS - 1)
    def _():
        pltpu.sync_copy(acc.at[pl.ds(N - 16, 16)],
                        msg_out.at[cid, pl.ds(N - 16, 16)])


def _mesh():
    return plsc.VectorSubcoreMesh(core_axis_name="c", subcore_axis_name="s",
                                  num_cores=NC, num_subcores=NS)


def _sc_call(table, src2d, dst2d):
    f = pl.kernel(
        _sc_body,
        out_type=[jax.ShapeDtypeStruct((NC, N, D), _f32)],
        mesh=_mesh(),
        scratch_types=[
            pltpu.VMEM((G, CH), jnp.int32),
            pltpu.VMEM((G, CH), jnp.int32),
            pltpu.VMEM((G, CH), jnp.int32),
            pltpu.VMEM((G, CH), jnp.int32),
            pltpu.VMEM((CH, D), _f32),
            pltpu.VMEM((CH, D), _f32),
            pltpu.VMEM((CH, D), _f32),
            pltpu.VMEM((CH, D), _f32),
            pltpu.VMEM((ZR, D), _f32),
            pltpu.VMEM_SHARED((N, D), _f32),
            pltpu.SemaphoreType.DMA,
            pltpu.SemaphoreType.DMA,
            pltpu.SemaphoreType.DMA,
            pltpu.SemaphoreType.DMA,
            pltpu.SemaphoreType.DMA,
            pltpu.SemaphoreType.DMA,
        ],
    )
    return f(table, src2d, dst2d)[0]


def _deg_body(dst1d, deg_out, dgf, degv):
    cid = lax.axis_index("c")
    sid = lax.axis_index("s")
    wid = cid * NS + sid

    @pl.loop(0, N // 16)
    def _(i):
        degv[pl.ds(i * 16, 16)] = jnp.zeros((16,), _f32)

    @pl.loop(0, NG)
    def _(g):
        off = pl.multiple_of(wid * EW + g * G * CH, 8)
        pltpu.sync_copy(dst1d.at[pl.ds(off, G * CH)], dgf)

        @pl.loop(0, G * CH // 16)
        def _(i):
            v = dgf[pl.ds(i * 16, 16)]
            plsc.addupdate_scatter(degv, [v], jnp.ones((16,), _f32))

    pltpu.sync_copy(degv, deg_out.at[wid])


def _deg_call(dst1d):
    import dataclasses as _dc
    cp = pltpu.CompilerParams()
    if "needs_layout_passes" in pltpu.CompilerParams.__dataclass_fields__:
        cp = _dc.replace(cp, needs_layout_passes=False)
    f = pl.kernel(
        _deg_body,
        out_type=[jax.ShapeDtypeStruct((NW, N), _f32)],
        mesh=_mesh(),
        compiler_params=cp,
        scratch_types=[
            pltpu.VMEM((G * CH,), jnp.int32),
            pltpu.VMEM((N,), _f32),
        ],
    )
    return f(dst1d)[0]


def _inv_deg(dg_block):
    deg = jnp.clip(jnp.sum(dg_block, axis=1, keepdims=True), 1.0, None)
    return 1.0 / deg


def _l1_body(z_ref, mp_ref, dg_ref, ws_ref, wn_ref, h1_ref):
    mp = mp_ref[...]
    msg = (mp[0] + mp[1]) * _inv_deg(dg_ref[...])
    h1_ref[...] = jnp.maximum(
        _dot(z_ref[...], ws_ref[...]) + _dot(msg, wn_ref[...]), 0.0)


def _l1_call(z, mp, dg, ws, wn):
    full = lambda s: pl.BlockSpec(s, lambda i: (0,) * len(s))
    return pl.pallas_call(
        _l1_body,
        grid=(GRID,),
        in_specs=[
            pl.BlockSpec((ROWS, D), lambda i: (i, 0)),
            pl.BlockSpec((NC, ROWS, D), lambda i: (0, i, 0)),
            pl.BlockSpec((ROWS, NW), lambda i: (i, 0)),
            full((D, D)), full((D, D)),
        ],
        out_specs=pl.BlockSpec((ROWS, D), lambda i: (i, 0)),
        out_shape=jax.ShapeDtypeStruct((N, D), _f32),
    )(z, mp, dg, ws, wn)


def _l2_body(h1_ref, mp_ref, dg_ref, ws_ref, wn_ref, wh_ref, bh_ref,
             out_ref):
    mp = mp_ref[...]
    msg = (mp[0] + mp[1]) * _inv_deg(dg_ref[...])
    h2 = _dot(h1_ref[...], ws_ref[...]) + _dot(msg, wn_ref[...])
    out_ref[...] = _dot(h2, wh_ref[...]) + bh_ref[...]


def _l2_call(h1, mp, dg, ws, wn, wh, bh):
    full = lambda s: pl.BlockSpec(s, lambda i: (0,) * len(s))
    return pl.pallas_call(
        _l2_body,
        grid=(GRID,),
        in_specs=[
            pl.BlockSpec((ROWS, D), lambda i: (i, 0)),
            pl.BlockSpec((NC, ROWS, D), lambda i: (0, i, 0)),
            pl.BlockSpec((ROWS, NW), lambda i: (i, 0)),
            full((D, D)), full((D, D)), full((D, OUT)), full((1, OUT)),
        ],
        out_specs=pl.BlockSpec((ROWS, OUT), lambda i: (i, 0)),
        out_shape=jax.ShapeDtypeStruct((N, OUT), _f32),
    )(h1, mp, dg, ws, wn, wh, bh)


def kernel(x, edge_index, type_logit, col_logit, vib_mu_W, vib_mu_b,
           vib_lv_W, vib_lv_b, vib_eps, W_self1, W_neigh1, W_self2,
           W_neigh2, W_head, b_head):
    src2d = edge_index[0].reshape(NW * NCH, CH)
    dst1d = edge_index[1]
    dst2d = dst1d.reshape(NW * NCH, CH)
    z, kl11 = _vib_call(x, vib_eps, type_logit.reshape(1, 1),
                        col_logit.reshape(1, D), vib_mu_W,
                        vib_mu_b.reshape(1, D), vib_lv_W,
                        vib_lv_b.reshape(1, D))
    degp = _deg_call(dst1d)
    msg1p = _sc_call(z, src2d, dst2d)
    degT = degp.T
    h1 = _l1_call(z, msg1p, degT, W_self1, W_neigh1)
    msg2p = _sc_call(h1, src2d, dst2d)
    out = _l2_call(h1, msg2p, degT, W_self2, W_neigh2, W_head,
                   b_head.reshape(1, OUT))
    return (out, kl11[0, 0])
